# baseline (speedup 1.0000x reference)
import jax
import jax.numpy as jnp
from jax.experimental import pallas as pl
from jax.experimental.pallas import tpu as pltpu

S, B, D = 2048, 4, 2048
NCH = 8
CH = S // NCH


def _concat_embed_body(t_ref, x_ref, table_ref, out_ref, copy_sems, gat_sems):
    copies = []
    for i in range(NCH):
        c = pltpu.make_async_copy(
            x_ref.at[pl.ds(i * CH, CH)],
            out_ref.at[pl.ds(i * CH, CH)],
            copy_sems.at[i],
        )
        c.start()
        copies.append(c)
    gathers = []
    for b in range(B):
        g = pltpu.make_async_copy(
            table_ref.at[t_ref[b]],
            out_ref.at[S, b],
            gat_sems.at[b],
        )
        g.start()
        gathers.append(g)
    for c in copies:
        c.wait()
    for g in gathers:
        g.wait()


def kernel(x, t, table):
    return pl.pallas_call(
        _concat_embed_body,
        out_shape=jax.ShapeDtypeStruct((S + 1, B, D), x.dtype),
        in_specs=[
            pl.BlockSpec(memory_space=pltpu.SMEM),
            pl.BlockSpec(memory_space=pl.ANY),
            pl.BlockSpec(memory_space=pl.ANY),
        ],
        out_specs=pl.BlockSpec(memory_space=pl.ANY),
        scratch_shapes=[
            pltpu.SemaphoreType.DMA((NCH,)),
            pltpu.SemaphoreType.DMA((B,)),
        ],
    )(t, x, table)


# speedup vs baseline: 47.0343x; 47.0343x over previous
import jax
import jax.numpy as jnp
from jax.experimental import pallas as pl
from jax.experimental.pallas import tpu as pltpu

S, B, D = 2048, 4, 2048
BS = 128
N = S // BS


def _concat_embed_body(t_ref, x_ref, table_ref, out_ref, gat_sems):
    i = pl.program_id(0)

    @pl.when(i < N)
    def _copy():
        out_ref[...] = x_ref[...]

    @pl.when(i == N)
    def _embed():
        gathers = []
        for b in range(B):
            g = pltpu.make_async_copy(
                table_ref.at[t_ref[b]],
                out_ref.at[0, b],
                gat_sems.at[b],
            )
            g.start()
            gathers.append(g)
        for g in gathers:
            g.wait()


def kernel(x, t, table):
    return pl.pallas_call(
        _concat_embed_body,
        grid=(N + 1,),
        out_shape=jax.ShapeDtypeStruct((S + 1, B, D), x.dtype),
        in_specs=[
            pl.BlockSpec(memory_space=pltpu.SMEM),
            pl.BlockSpec((BS, B, D), lambda i: (jnp.minimum(i, N - 1), 0, 0)),
            pl.BlockSpec(memory_space=pl.ANY),
        ],
        out_specs=pl.BlockSpec((BS, B, D), lambda i: (i, 0, 0)),
        scratch_shapes=[
            pltpu.SemaphoreType.DMA((B,)),
        ],
    )(t, x, table)
